# Initial kernel scaffold; baseline (speedup 1.0000x reference)
import jax
import jax.numpy as jnp
from jax.experimental import pallas as pl


def kernel(x, edge_index, W1, b1, W2, b2):
    raise NotImplementedError("write your pallas kernel here")


# speedup vs baseline: 60.2449x; 60.2449x over previous
import functools

import jax
import jax.numpy as jnp
from jax import lax
from jax.experimental import pallas as pl
from jax.experimental.pallas import tpu as pltpu
from jax.experimental.pallas import tpu_sc as plsc

N = 100000
NP = 100096
STRIPE = NP // 16
E = 6400000
LANES = 128
G = 8
NCORES = 2
NSUB = 16
TILES = NCORES * NSUB
OUTER = 196
ROWS = TILES * OUTER * G
EPAD = ROWS * LANES
ROWS_PER_TILE = OUTER * G


def _mesh():
    return plsc.VectorSubcoreMesh(core_axis_name="c", subcore_axis_name="s")


def _hist_body(dst_hbm, zero_hbm, out_hbm, idx_v, ones_v, stage, deg_sh):
    c = lax.axis_index("c")
    s = lax.axis_index("s")
    wid = c * NSUB + s
    pltpu.sync_copy(zero_hbm.at[pl.ds(s * STRIPE, STRIPE)], stage)
    pltpu.sync_copy(stage, deg_sh.at[pl.ds(s * STRIPE, STRIPE)])
    for i in range(LANES // 16):
        ones_v[pl.ds(i * 16, 16)] = jnp.ones((16,), jnp.float32)
    plsc.subcore_barrier()

    base = wid * ROWS_PER_TILE

    def step(o, carry):
        r0 = base + o * G
        pltpu.sync_copy(dst_hbm.at[pl.ds(r0, G)], idx_v)
        for j in range(G):
            pltpu.sync_copy(ones_v, deg_sh.at[idx_v.at[j]], add=True)
        return carry

    lax.fori_loop(0, OUTER, step, 0)
    plsc.subcore_barrier()
    pltpu.sync_copy(deg_sh.at[pl.ds(s * STRIPE, STRIPE)], stage)
    pltpu.sync_copy(stage, out_hbm.at[pl.ds(c * NP + s * STRIPE, STRIPE)])


def _make_hist():
    return pl.kernel(
        _hist_body,
        out_type=jax.ShapeDtypeStruct((NCORES * NP,), jnp.float32),
        mesh=_mesh(),
        scratch_types=[
            pltpu.VMEM((G, LANES), jnp.int32),
            pltpu.VMEM((LANES,), jnp.float32),
            pltpu.VMEM((STRIPE,), jnp.float32),
            pltpu.VMEM_SHARED((NP,), jnp.float32),
        ],
    )


def _agg_body(src_hbm, dst_hbm, tab_hbm, zero_hbm, out_hbm,
              sidx, didx, rows, stage, acc_sh, gsem):
    c = lax.axis_index("c")
    s = lax.axis_index("s")
    wid = c * NSUB + s
    pltpu.sync_copy(zero_hbm.at[pl.ds(s * STRIPE, STRIPE), :], stage)
    pltpu.sync_copy(stage, acc_sh.at[pl.ds(s * STRIPE, STRIPE), :])
    plsc.subcore_barrier()

    base = wid * ROWS_PER_TILE

    def step(o, carry):
        r0 = base + o * G
        pltpu.sync_copy(src_hbm.at[pl.ds(r0, G)], sidx)
        pltpu.sync_copy(dst_hbm.at[pl.ds(r0, G)], didx)
        descs = [pltpu.async_copy(tab_hbm.at[sidx.at[j]], rows.at[j], gsem)
                 for j in range(G)]
        for d in descs:
            d.wait()
        for j in range(G):
            pltpu.sync_copy(rows.at[j], acc_sh.at[didx.at[j]], add=True)
        return carry

    lax.fori_loop(0, OUTER, step, 0)
    plsc.subcore_barrier()
    pltpu.sync_copy(acc_sh.at[pl.ds(s * STRIPE, STRIPE), :], stage)
    pltpu.sync_copy(stage, out_hbm.at[c, pl.ds(s * STRIPE, STRIPE), :])


def _make_agg(width):
    return pl.kernel(
        _agg_body,
        out_type=jax.ShapeDtypeStruct((NCORES, NP, width), jnp.float32),
        mesh=_mesh(),
        scratch_types=[
            pltpu.VMEM((G, LANES), jnp.int32),
            pltpu.VMEM((G, LANES), jnp.int32),
            pltpu.VMEM((G, LANES, width), jnp.float32),
            pltpu.VMEM((STRIPE, width), jnp.float32),
            pltpu.VMEM_SHARED((NP, width), jnp.float32),
            pltpu.SemaphoreType.DMA,
        ],
        compiler_params=pltpu.CompilerParams(use_tc_tiling_on_sc=False),
    )


def _tc1_body(d0, d1, x, dis_ref, inv_ref, gx_ref):
    deg = d0[...] + d1[...] + 1.0
    inv = 1.0 / deg
    dis = lax.rsqrt(deg)
    dis_ref[...] = dis
    inv_ref[...] = inv
    gx_ref[...] = jnp.concatenate([x[...] * dis, jnp.zeros((STRIPE, 4), jnp.float32)], axis=1)


def _tc2_body(a0, a1, x, dis, inv, w1, b1, w2, b2, gp_ref, ps_ref):
    agg1 = (a0[...][:, :4] + a1[...][:, :4]) * dis[...] + x[...] * inv[...]
    h = jnp.dot(agg1, w1[...], preferred_element_type=jnp.float32) + b1[...]
    h = jnp.maximum(h, 0.0)
    p = jnp.dot(h, w2[...], preferred_element_type=jnp.float32)
    i = pl.program_id(0)
    rid = i * STRIPE + lax.broadcasted_iota(jnp.int32, (STRIPE, 1), 0)
    gp = jnp.where(rid < N, p * dis[...], 0.0)
    gp_ref[...] = jnp.concatenate([gp, jnp.zeros((STRIPE, 6), jnp.float32)], axis=1)
    ps_ref[...] = p * inv[...] + b2[...]


def _tc3_body(g0, g1, dis, ps, out_ref):
    out_ref[...] = (g0[...][:, :2] + g1[...][:, :2]) * dis[...] + ps[...]


def _col_spec(width):
    return pl.BlockSpec((STRIPE, width), lambda i: (i, 0))


def _full_spec():
    return pl.BlockSpec(index_map=lambda i: (0, 0))


def kernel(x, edge_index, W1, b1, W2, b2):
    src = edge_index[0].astype(jnp.int32)
    dst = edge_index[1].astype(jnp.int32)
    pad = jnp.full((EPAD - E,), N, dtype=jnp.int32)
    src_r = jnp.concatenate([src, pad]).reshape(ROWS, LANES)
    dst_r = jnp.concatenate([dst, pad]).reshape(ROWS, LANES)
    xp = jnp.concatenate([x, jnp.zeros((NP - N, 4), jnp.float32)], axis=0)

    zero1 = jnp.zeros((NP,), jnp.float32)
    zero8 = jnp.zeros((NP, 8), jnp.float32)

    degp = _make_hist()(dst_r, zero1).reshape(NCORES, NP)
    d0 = degp[0].reshape(NP, 1)
    d1 = degp[1].reshape(NP, 1)

    dis, inv, gx = pl.pallas_call(
        _tc1_body,
        grid=(16,),
        in_specs=[_col_spec(1), _col_spec(1), _col_spec(4)],
        out_specs=[_col_spec(1), _col_spec(1), _col_spec(8)],
        out_shape=[
            jax.ShapeDtypeStruct((NP, 1), jnp.float32),
            jax.ShapeDtypeStruct((NP, 1), jnp.float32),
            jax.ShapeDtypeStruct((NP, 8), jnp.float32),
        ],
    )(d0, d1, xp)

    a = _make_agg(8)(src_r, dst_r, gx, zero8)

    gp, ps = pl.pallas_call(
        _tc2_body,
        grid=(16,),
        in_specs=[_col_spec(8), _col_spec(8), _col_spec(4), _col_spec(1),
                  _col_spec(1), _full_spec(), _full_spec(), _full_spec(),
                  _full_spec()],
        out_specs=[_col_spec(8), _col_spec(2)],
        out_shape=[
            jax.ShapeDtypeStruct((NP, 8), jnp.float32),
            jax.ShapeDtypeStruct((NP, 2), jnp.float32),
        ],
    )(a[0], a[1], xp, dis, inv, W1, b1.reshape(1, 16), W2, b2.reshape(1, 2))

    g = _make_agg(8)(src_r, dst_r, gp, zero8)

    out = pl.pallas_call(
        _tc3_body,
        grid=(16,),
        in_specs=[_col_spec(8), _col_spec(8), _col_spec(1), _col_spec(2)],
        out_specs=_col_spec(2),
        out_shape=jax.ShapeDtypeStruct((NP, 2), jnp.float32),
    )(g[0], g[1], dis, ps)

    return out[:N]


# speedup vs baseline: 63.8315x; 1.0595x over previous
import functools

import jax
import jax.numpy as jnp
from jax import lax
from jax.experimental import pallas as pl
from jax.experimental.pallas import tpu as pltpu
from jax.experimental.pallas import tpu_sc as plsc

N = 100000
NP = 100096
STRIPE = NP // 16
E = 6400000
LANES = 128
G = 8
NCORES = 2
NSUB = 16
TILES = NCORES * NSUB
OUTER = 196
ROWS = TILES * OUTER * G
EPAD = ROWS * LANES
ROWS_PER_TILE = OUTER * G


def _mesh():
    return plsc.VectorSubcoreMesh(core_axis_name="c", subcore_axis_name="s")


def _hist_body(dst_hbm, zero_hbm, out_hbm, idx_v, ones_v, stage, deg_sh):
    c = lax.axis_index("c")
    s = lax.axis_index("s")
    wid = c * NSUB + s
    pltpu.sync_copy(zero_hbm.at[pl.ds(s * STRIPE, STRIPE)], stage)
    pltpu.sync_copy(stage, deg_sh.at[pl.ds(s * STRIPE, STRIPE)])
    for i in range(LANES // 16):
        ones_v[pl.ds(i * 16, 16)] = jnp.ones((16,), jnp.float32)
    plsc.subcore_barrier()

    base = wid * ROWS_PER_TILE

    def step(o, carry):
        r0 = base + o * G
        pltpu.sync_copy(dst_hbm.at[pl.ds(r0, G)], idx_v)
        for j in range(G):
            pltpu.sync_copy(ones_v, deg_sh.at[idx_v.at[j]], add=True)
        return carry

    lax.fori_loop(0, OUTER, step, 0)
    plsc.subcore_barrier()
    pltpu.sync_copy(deg_sh.at[pl.ds(s * STRIPE, STRIPE)], stage)
    pltpu.sync_copy(stage, out_hbm.at[pl.ds(c * NP + s * STRIPE, STRIPE)])


def _make_hist():
    return pl.kernel(
        _hist_body,
        out_type=jax.ShapeDtypeStruct((NCORES * NP,), jnp.float32),
        mesh=_mesh(),
        scratch_types=[
            pltpu.VMEM((G, LANES), jnp.int32),
            pltpu.VMEM((LANES,), jnp.float32),
            pltpu.VMEM((STRIPE,), jnp.float32),
            pltpu.VMEM_SHARED((NP,), jnp.float32),
        ],
    )


def _agg_body(src_hbm, dst_hbm, tab_hbm, zero_hbm, out_hbm,
              sidx, didx, rows, stage, acc_sh, gsem, ssem):
    c = lax.axis_index("c")
    s = lax.axis_index("s")
    wid = c * NSUB + s
    pltpu.sync_copy(zero_hbm.at[pl.ds(s * STRIPE, STRIPE), :], stage)
    pltpu.sync_copy(stage, acc_sh.at[pl.ds(s * STRIPE, STRIPE), :])
    plsc.subcore_barrier()

    base = wid * ROWS_PER_TILE

    def step(o, carry):
        r0 = base + o * G
        pltpu.sync_copy(src_hbm.at[pl.ds(r0, G)], sidx)
        pltpu.sync_copy(dst_hbm.at[pl.ds(r0, G)], didx)
        descs = [pltpu.async_copy(tab_hbm.at[sidx.at[j]], rows.at[j], gsem)
                 for j in range(G)]
        for d in descs:
            d.wait()
        sdescs = [pltpu.async_copy(rows.at[j], acc_sh.at[didx.at[j]], ssem,
                                   add=True)
                  for j in range(G)]
        for d in sdescs:
            d.wait()
        return carry

    lax.fori_loop(0, OUTER, step, 0)
    plsc.subcore_barrier()
    pltpu.sync_copy(acc_sh.at[pl.ds(s * STRIPE, STRIPE), :], stage)
    pltpu.sync_copy(stage, out_hbm.at[c, pl.ds(s * STRIPE, STRIPE), :])


def _make_agg(width):
    return pl.kernel(
        _agg_body,
        out_type=jax.ShapeDtypeStruct((NCORES, NP, width), jnp.float32),
        mesh=_mesh(),
        scratch_types=[
            pltpu.VMEM((G, LANES), jnp.int32),
            pltpu.VMEM((G, LANES), jnp.int32),
            pltpu.VMEM((G, LANES, width), jnp.float32),
            pltpu.VMEM((STRIPE, width), jnp.float32),
            pltpu.VMEM_SHARED((NP, width), jnp.float32),
            pltpu.SemaphoreType.DMA,
            pltpu.SemaphoreType.DMA,
        ],
        compiler_params=pltpu.CompilerParams(use_tc_tiling_on_sc=False),
    )


def _tc1_body(d0, d1, x, dis_ref, inv_ref, gx_ref):
    deg = d0[...] + d1[...] + 1.0
    inv = 1.0 / deg
    dis = lax.rsqrt(deg)
    dis_ref[...] = dis
    inv_ref[...] = inv
    gx_ref[...] = jnp.concatenate([x[...] * dis, jnp.zeros((STRIPE, 4), jnp.float32)], axis=1)


def _tc2_body(a0, a1, x, dis, inv, w1, b1, w2, b2, gp_ref, ps_ref):
    agg1 = (a0[...][:, :4] + a1[...][:, :4]) * dis[...] + x[...] * inv[...]
    h = jnp.dot(agg1, w1[...], preferred_element_type=jnp.float32) + b1[...]
    h = jnp.maximum(h, 0.0)
    p = jnp.dot(h, w2[...], preferred_element_type=jnp.float32)
    i = pl.program_id(0)
    rid = i * STRIPE + lax.broadcasted_iota(jnp.int32, (STRIPE, 1), 0)
    gp = jnp.where(rid < N, p * dis[...], 0.0)
    gp_ref[...] = jnp.concatenate([gp, jnp.zeros((STRIPE, 6), jnp.float32)], axis=1)
    ps_ref[...] = p * inv[...] + b2[...]


def _tc3_body(g0, g1, dis, ps, out_ref):
    out_ref[...] = (g0[...][:, :2] + g1[...][:, :2]) * dis[...] + ps[...]


def _col_spec(width):
    return pl.BlockSpec((STRIPE, width), lambda i: (i, 0))


def _full_spec():
    return pl.BlockSpec(index_map=lambda i: (0, 0))


def kernel(x, edge_index, W1, b1, W2, b2):
    src = edge_index[0].astype(jnp.int32)
    dst = edge_index[1].astype(jnp.int32)
    pad = jnp.full((EPAD - E,), N, dtype=jnp.int32)
    src_r = jnp.concatenate([src, pad]).reshape(ROWS, LANES)
    dst_r = jnp.concatenate([dst, pad]).reshape(ROWS, LANES)
    xp = jnp.concatenate([x, jnp.zeros((NP - N, 4), jnp.float32)], axis=0)

    zero1 = jnp.zeros((NP,), jnp.float32)
    zero8 = jnp.zeros((NP, 8), jnp.float32)

    degp = _make_hist()(dst_r, zero1).reshape(NCORES, NP)
    d0 = degp[0].reshape(NP, 1)
    d1 = degp[1].reshape(NP, 1)

    dis, inv, gx = pl.pallas_call(
        _tc1_body,
        grid=(16,),
        in_specs=[_col_spec(1), _col_spec(1), _col_spec(4)],
        out_specs=[_col_spec(1), _col_spec(1), _col_spec(8)],
        out_shape=[
            jax.ShapeDtypeStruct((NP, 1), jnp.float32),
            jax.ShapeDtypeStruct((NP, 1), jnp.float32),
            jax.ShapeDtypeStruct((NP, 8), jnp.float32),
        ],
    )(d0, d1, xp)

    a = _make_agg(8)(src_r, dst_r, gx, zero8)

    gp, ps = pl.pallas_call(
        _tc2_body,
        grid=(16,),
        in_specs=[_col_spec(8), _col_spec(8), _col_spec(4), _col_spec(1),
                  _col_spec(1), _full_spec(), _full_spec(), _full_spec(),
                  _full_spec()],
        out_specs=[_col_spec(8), _col_spec(2)],
        out_shape=[
            jax.ShapeDtypeStruct((NP, 8), jnp.float32),
            jax.ShapeDtypeStruct((NP, 2), jnp.float32),
        ],
    )(a[0], a[1], xp, dis, inv, W1, b1.reshape(1, 16), W2, b2.reshape(1, 2))

    g = _make_agg(8)(src_r, dst_r, gp, zero8)

    out = pl.pallas_call(
        _tc3_body,
        grid=(16,),
        in_specs=[_col_spec(8), _col_spec(8), _col_spec(1), _col_spec(2)],
        out_specs=_col_spec(2),
        out_shape=jax.ShapeDtypeStruct((NP, 2), jnp.float32),
    )(g[0], g[1], dis, ps)

    return out[:N]


# speedup vs baseline: 72.6704x; 1.1385x over previous
import functools

import jax
import jax.numpy as jnp
from jax import lax
from jax.experimental import pallas as pl
from jax.experimental.pallas import tpu as pltpu
from jax.experimental.pallas import tpu_sc as plsc

N = 100000
NP = 100096
STRIPE = NP // 16
E = 6400000
LANES = 128
G = 16
NCORES = 2
NSUB = 16
TILES = NCORES * NSUB
OUTER = 98
ROWS = TILES * OUTER * G
EPAD = ROWS * LANES
ROWS_PER_TILE = OUTER * G


def _mesh():
    return plsc.VectorSubcoreMesh(core_axis_name="c", subcore_axis_name="s")


def _hist_body(dst_hbm, zero_hbm, out_hbm, idx_v, ones_v, stage, deg_sh):
    c = lax.axis_index("c")
    s = lax.axis_index("s")
    wid = c * NSUB + s
    pltpu.sync_copy(zero_hbm.at[pl.ds(s * STRIPE, STRIPE)], stage)
    pltpu.sync_copy(stage, deg_sh.at[pl.ds(s * STRIPE, STRIPE)])
    for i in range(LANES // 16):
        ones_v[pl.ds(i * 16, 16)] = jnp.ones((16,), jnp.float32)
    plsc.subcore_barrier()

    base = wid * ROWS_PER_TILE

    def step(o, carry):
        r0 = base + o * G
        pltpu.sync_copy(dst_hbm.at[pl.ds(r0, G)], idx_v)
        for j in range(G):
            pltpu.sync_copy(ones_v, deg_sh.at[idx_v.at[j]], add=True)
        return carry

    lax.fori_loop(0, OUTER, step, 0)
    plsc.subcore_barrier()
    pltpu.sync_copy(deg_sh.at[pl.ds(s * STRIPE, STRIPE)], stage)
    pltpu.sync_copy(stage, out_hbm.at[pl.ds(c * NP + s * STRIPE, STRIPE)])


def _make_hist():
    return pl.kernel(
        _hist_body,
        out_type=jax.ShapeDtypeStruct((NCORES * NP,), jnp.float32),
        mesh=_mesh(),
        scratch_types=[
            pltpu.VMEM((G, LANES), jnp.int32),
            pltpu.VMEM((LANES,), jnp.float32),
            pltpu.VMEM((STRIPE,), jnp.float32),
            pltpu.VMEM_SHARED((NP,), jnp.float32),
        ],
    )


def _agg_body(src_hbm, dst_hbm, tab_hbm, zero_hbm, out_hbm,
              sidx, didx, rows, stage, acc_sh, gsem, ssem):
    c = lax.axis_index("c")
    s = lax.axis_index("s")
    wid = c * NSUB + s
    pltpu.sync_copy(zero_hbm.at[pl.ds(s * STRIPE, STRIPE), :], stage)
    pltpu.sync_copy(stage, acc_sh.at[pl.ds(s * STRIPE, STRIPE), :])
    plsc.subcore_barrier()

    base = wid * ROWS_PER_TILE

    def step(o, carry):
        r0 = base + o * G
        pltpu.sync_copy(src_hbm.at[pl.ds(r0, G)], sidx)
        pltpu.sync_copy(dst_hbm.at[pl.ds(r0, G)], didx)
        descs = [pltpu.async_copy(tab_hbm.at[sidx.at[j]], rows.at[j], gsem)
                 for j in range(G)]
        for d in descs:
            d.wait()
        sdescs = [pltpu.async_copy(rows.at[j], acc_sh.at[didx.at[j]], ssem,
                                   add=True)
                  for j in range(G)]
        for d in sdescs:
            d.wait()
        return carry

    lax.fori_loop(0, OUTER, step, 0)
    plsc.subcore_barrier()
    pltpu.sync_copy(acc_sh.at[pl.ds(s * STRIPE, STRIPE), :], stage)
    pltpu.sync_copy(stage, out_hbm.at[c, pl.ds(s * STRIPE, STRIPE), :])


def _make_agg(width):
    return pl.kernel(
        _agg_body,
        out_type=jax.ShapeDtypeStruct((NCORES, NP, width), jnp.float32),
        mesh=_mesh(),
        scratch_types=[
            pltpu.VMEM((G, LANES), jnp.int32),
            pltpu.VMEM((G, LANES), jnp.int32),
            pltpu.VMEM((G, LANES, width), jnp.float32),
            pltpu.VMEM((STRIPE, width), jnp.float32),
            pltpu.VMEM_SHARED((NP, width), jnp.float32),
            pltpu.SemaphoreType.DMA,
            pltpu.SemaphoreType.DMA,
        ],
        compiler_params=pltpu.CompilerParams(use_tc_tiling_on_sc=False),
    )


def _tc1_body(d0, d1, x, dis_ref, inv_ref, gx_ref):
    deg = d0[...] + d1[...] + 1.0
    inv = 1.0 / deg
    dis = lax.rsqrt(deg)
    dis_ref[...] = dis
    inv_ref[...] = inv
    gx_ref[...] = jnp.concatenate([x[...] * dis, jnp.zeros((STRIPE, 4), jnp.float32)], axis=1)


def _tc2_body(a0, a1, x, dis, inv, w1, b1, w2, b2, gp_ref, ps_ref):
    agg1 = (a0[...][:, :4] + a1[...][:, :4]) * dis[...] + x[...] * inv[...]
    h = jnp.dot(agg1, w1[...], preferred_element_type=jnp.float32) + b1[...]
    h = jnp.maximum(h, 0.0)
    p = jnp.dot(h, w2[...], preferred_element_type=jnp.float32)
    i = pl.program_id(0)
    rid = i * STRIPE + lax.broadcasted_iota(jnp.int32, (STRIPE, 1), 0)
    gp = jnp.where(rid < N, p * dis[...], 0.0)
    gp_ref[...] = jnp.concatenate([gp, jnp.zeros((STRIPE, 6), jnp.float32)], axis=1)
    ps_ref[...] = p * inv[...] + b2[...]


def _tc3_body(g0, g1, dis, ps, out_ref):
    out_ref[...] = (g0[...][:, :2] + g1[...][:, :2]) * dis[...] + ps[...]


def _col_spec(width):
    return pl.BlockSpec((STRIPE, width), lambda i: (i, 0))


def _full_spec():
    return pl.BlockSpec(index_map=lambda i: (0, 0))


def kernel(x, edge_index, W1, b1, W2, b2):
    src = edge_index[0].astype(jnp.int32)
    dst = edge_index[1].astype(jnp.int32)
    pad = jnp.full((EPAD - E,), N, dtype=jnp.int32)
    src_r = jnp.concatenate([src, pad]).reshape(ROWS, LANES)
    dst_r = jnp.concatenate([dst, pad]).reshape(ROWS, LANES)
    xp = jnp.concatenate([x, jnp.zeros((NP - N, 4), jnp.float32)], axis=0)

    zero1 = jnp.zeros((NP,), jnp.float32)
    zero8 = jnp.zeros((NP, 8), jnp.float32)

    degp = _make_hist()(dst_r, zero1).reshape(NCORES, NP)
    d0 = degp[0].reshape(NP, 1)
    d1 = degp[1].reshape(NP, 1)

    dis, inv, gx = pl.pallas_call(
        _tc1_body,
        grid=(16,),
        in_specs=[_col_spec(1), _col_spec(1), _col_spec(4)],
        out_specs=[_col_spec(1), _col_spec(1), _col_spec(8)],
        out_shape=[
            jax.ShapeDtypeStruct((NP, 1), jnp.float32),
            jax.ShapeDtypeStruct((NP, 1), jnp.float32),
            jax.ShapeDtypeStruct((NP, 8), jnp.float32),
        ],
    )(d0, d1, xp)

    a = _make_agg(8)(src_r, dst_r, gx, zero8)

    gp, ps = pl.pallas_call(
        _tc2_body,
        grid=(16,),
        in_specs=[_col_spec(8), _col_spec(8), _col_spec(4), _col_spec(1),
                  _col_spec(1), _full_spec(), _full_spec(), _full_spec(),
                  _full_spec()],
        out_specs=[_col_spec(8), _col_spec(2)],
        out_shape=[
            jax.ShapeDtypeStruct((NP, 8), jnp.float32),
            jax.ShapeDtypeStruct((NP, 2), jnp.float32),
        ],
    )(a[0], a[1], xp, dis, inv, W1, b1.reshape(1, 16), W2, b2.reshape(1, 2))

    g = _make_agg(8)(src_r, dst_r, gp, zero8)

    out = pl.pallas_call(
        _tc3_body,
        grid=(16,),
        in_specs=[_col_spec(8), _col_spec(8), _col_spec(1), _col_spec(2)],
        out_specs=_col_spec(2),
        out_shape=jax.ShapeDtypeStruct((NP, 2), jnp.float32),
    )(g[0], g[1], dis, ps)

    return out[:N]
